# baseline (speedup 1.0000x reference)
import jax
import jax.numpy as jnp
from jax.experimental import pallas as pl
from jax.experimental.pallas import tpu as pltpu

B, S, D = 2, 2048, 1024
E, K = 8, 2
H = 2 * D
AD = H // 16
T = B * S

BLK1 = 1024
BLK2 = 1024

_bf16 = jnp.bfloat16
_f32 = jnp.float32


def _lnk(h, g, b, eps=1e-5):
    m = jnp.mean(h, axis=-1, keepdims=True)
    v = jnp.mean((h - m) * (h - m), axis=-1, keepdims=True)
    return (h - m) * jax.lax.rsqrt(v + eps) * g + b


def _dot_t(a, w):
    return jax.lax.dot_general(a, w, (((1,), (1,)), ((), ())),
                               preferred_element_type=_f32)


def _prep_kernel(wdown_ref, wadapt_ref, wout_ref, wexp_ref, wda_ref, wcomb_ref):
    wd = wdown_ref[...].astype(_bf16)
    wa = wadapt_ref[...].astype(_bf16)
    wo = wout_ref[...].astype(_bf16)
    we = wexp_ref[...].astype(_bf16)
    mm = lambda a, b: jax.lax.dot_general(
        a, b, (((1,), (0,)), ((), ())), preferred_element_type=_f32)
    wda_ref[...] = (0.1 * mm(wd, wa)).astype(_bf16)
    wcomb_ref[...] = mm(wo, we).astype(_bf16)


def _stage1_kernel(x_ref, wup_ref, wgate_ref, wpre_ref, wpost_ref, wr_ref,
                   acat_ref, gr_ref, br_ref, temp_ref, bup_ref, bgate_ref,
                   bpre_ref, bpost_ref, ga_ref, ba_ref, gexp_ref, bexp_ref,
                   hidden_ref, ai_ref, ao_ref, mixed_ref, stats_ref):
    xb = x_ref[...]
    xbf = xb.astype(_bf16)

    up = _dot_t(xbf, wup_ref[...]) + bup_ref[...]
    gate = _dot_t(xbf, wgate_ref[...]) + bgate_ref[...]
    hidden = jax.nn.silu(gate) * up
    hidden_bf = hidden.astype(_bf16)
    hidden_ref[...] = hidden_bf

    pre = _dot_t(xbf, wpre_ref[...])
    ga = ga_ref[...]
    ba = ba_ref[...]
    adapt_in = _lnk(pre + bpre_ref[...], ga, ba)
    a_ln = _lnk(pre, ga, ba)
    adapt_out = _lnk(_dot_t(hidden_bf, wpost_ref[...]) + bpost_ref[...], ga, ba)
    ai_ref[...] = adapt_in.astype(_bf16)
    ao_ref[...] = adapt_out.astype(_bf16)

    raw_t = jax.lax.dot_general(wr_ref[...], xb, (((1,), (1,)), ((), ())),
                                preferred_element_type=_f32,
                                precision=jax.lax.Precision.HIGHEST)
    m = jnp.mean(raw_t, axis=0, keepdims=True)
    v = jnp.mean((raw_t - m) * (raw_t - m), axis=0, keepdims=True)
    logits_t = (raw_t - m) * jax.lax.rsqrt(v + 1e-5) * gr_ref[...] + br_ref[...]
    z = logits_t / (temp_ref[0, 0] + 1e-6)
    z = z - jnp.max(z, axis=0, keepdims=True)
    ez = jnp.exp(z)
    rw_t = ez / jnp.sum(ez, axis=0, keepdims=True)

    neg = jnp.full((1, BLK1), -jnp.inf, _f32)
    m1 = neg
    m2 = neg
    i1 = jnp.zeros((1, BLK1), jnp.int32)
    i2 = jnp.zeros((1, BLK1), jnp.int32)
    for e in range(E):
        ve = rw_t[e:e + 1, :]
        gt1 = ve > m1
        gt2 = ve > m2
        i2 = jnp.where(gt1, i1, jnp.where(gt2, e, i2))
        m2 = jnp.where(gt1, m1, jnp.where(gt2, ve, m2))
        i1 = jnp.where(gt1, e, i1)
        m1 = jnp.where(gt1, ve, m1)
    eids = jax.lax.broadcasted_iota(jnp.int32, (E, BLK1), 0)
    dispatch_t = (jnp.where(eids == i1, m1, 0.0)
                  + jnp.where(eids == i2, m2, 0.0))

    r8 = (jax.lax.broadcasted_iota(jnp.int32, (E, E * AD), 1) // AD
          == jax.lax.broadcasted_iota(jnp.int32, (E, E * AD), 0))
    r8 = r8.astype(_bf16)
    mavg = (jax.lax.broadcasted_iota(jnp.int32, (E * AD, E), 0) // AD
            == jax.lax.broadcasted_iota(jnp.int32, (E * AD, E), 1))
    mavg = mavg.astype(_bf16) * (1.0 / AD)
    rsum = (jax.lax.broadcasted_iota(jnp.int32, (E * AD, AD), 0) % AD
            == jax.lax.broadcasted_iota(jnp.int32, (E * AD, AD), 1))
    rsum = rsum.astype(_bf16)

    mm = lambda a, b: jax.lax.dot_general(
        a, b, (((1,), (0,)), ((), ())), preferred_element_type=_f32)
    abf = a_ln.astype(_bf16)
    h_all = mm(abf, acat_ref[...])
    mean8 = mm(h_all.astype(_bf16), mavg)
    ex28 = mm((h_all * h_all).astype(_bf16), mavg)
    mean_full = mm(mean8.astype(_bf16), r8)
    ex2_full = mm(ex28.astype(_bf16), r8)
    rinv = jax.lax.rsqrt(jnp.maximum(ex2_full - mean_full * mean_full, 0.0)
                         + 1e-5)
    hl_all = (h_all - mean_full) * rinv * gexp_ref[...] + bexp_ref[...]
    dfull = jax.lax.dot_general(dispatch_t.astype(_bf16), r8,
                                (((0,), (0,)), ((), ())),
                                preferred_element_type=_f32)
    mixed = mm((hl_all * dfull).astype(_bf16), rsum)
    mixed_ref[...] = mixed.astype(_bf16)

    col8 = jnp.sum(dispatch_t, axis=1, keepdims=True)
    zp = jnp.sum(logits_t * logits_t)
    li = jax.lax.broadcasted_iota(jnp.int32, (E, 128), 1)
    si = jax.lax.broadcasted_iota(jnp.int32, (E, 128), 0)
    srow = (jnp.where(li == 0, col8, 0.0)
            + jnp.where((li == 1) & (si == 0), zp, 0.0)).reshape(1, E, 128)
    i = pl.program_id(0)

    @pl.when(i % (S // BLK1) == 0)
    def _init():
        stats_ref[...] = srow

    @pl.when(i % (S // BLK1) != 0)
    def _acc():
        stats_ref[...] += srow


def _stage2_kernel(hid_ref, x_ref, aiall_ref, aoall_ref, aib_ref, mixed_ref,
                   wdown_ref, wda_ref, wcomb_ref, bdown_ref, stats_ref,
                   y_ref, rloss_ref):
    ai_b = aib_ref[...]
    aw = _dot_t(ai_b, aoall_ref[...]).astype(_bf16)
    aw = jax.nn.silu(jnp.clip(aw, _bf16(-5.0), _bf16(5.0)))
    adapt = jax.lax.dot_general(aw, aiall_ref[...],
                                (((1,), (0,)), ((), ())),
                                preferred_element_type=_f32)

    y = (x_ref[...] + bdown_ref[...]
         + _dot_t(hid_ref[...], wdown_ref[...])
         + _dot_t(adapt.astype(_bf16), wda_ref[...])
         + _dot_t(mixed_ref[...], wcomb_ref[...]))
    y_ref[...] = y

    @pl.when(pl.program_id(0) == 0)
    def _loss():
        st = stats_ref[...]
        loads = st[:, :, 0] * (1.0 / S)
        zsum = jnp.sum(st[:, 0:1, 1:2])
        mean_l = jnp.mean(loads)
        var = jnp.sum((loads - mean_l) * (loads - mean_l)) / (B * E - 1)
        lb = jnp.sqrt(var) / mean_l * 10.0
        zl = zsum * (1.0 / (T * E))
        val = 0.001 * zl + 0.1 * lb
        rloss_ref[...] = jnp.full((1, 128), val, _f32)


def kernel(x, W_router, g_router, b_router, temperature, W_up, b_up, W_gate,
           b_gate, W_down, b_down, W_pre, b_pre, W_post, b_post, g_adapt,
           b_adapt, W_adapt_proj, A_experts, g_exp, b_exp, W_expert_proj,
           W_output_proj):
    xf = x.reshape(T, D)
    r2 = lambda v: v.reshape(1, -1)

    wda, wcomb = pl.pallas_call(
        _prep_kernel,
        out_shape=[jax.ShapeDtypeStruct((D, AD), _bf16),
                   jax.ShapeDtypeStruct((D, AD), _bf16)],
    )(W_down, W_adapt_proj, W_output_proj, W_expert_proj)

    n1 = T // BLK1
    c0 = lambda i: (0, 0)
    c000 = lambda i: (0, 0, 0)
    row_map = lambda i: (i, 0)
    hidden, ai, ao, mixed, stats = pl.pallas_call(
        _stage1_kernel,
        grid=(n1,),
        in_specs=[
            pl.BlockSpec((BLK1, D), row_map),
            pl.BlockSpec((H, D), c0),
            pl.BlockSpec((H, D), c0),
            pl.BlockSpec((AD, D), c0),
            pl.BlockSpec((AD, H), c0),
            pl.BlockSpec((E, D), c0),
            pl.BlockSpec((AD, E * AD), c0),
            pl.BlockSpec((E, 1), c0),
            pl.BlockSpec((E, 1), c0),
            pl.BlockSpec((1, 1), c0),
            pl.BlockSpec((1, H), c0),
            pl.BlockSpec((1, H), c0),
            pl.BlockSpec((1, AD), c0),
            pl.BlockSpec((1, AD), c0),
            pl.BlockSpec((1, AD), c0),
            pl.BlockSpec((1, AD), c0),
            pl.BlockSpec((1, E * AD), c0),
            pl.BlockSpec((1, E * AD), c0),
        ],
        out_specs=[
            pl.BlockSpec((BLK1, H), row_map),
            pl.BlockSpec((BLK1, AD), row_map),
            pl.BlockSpec((BLK1, AD), row_map),
            pl.BlockSpec((BLK1, AD), row_map),
            pl.BlockSpec((1, E, 128), lambda i: (i // (S // BLK1), 0, 0)),
        ],
        out_shape=[
            jax.ShapeDtypeStruct((T, H), _bf16),
            jax.ShapeDtypeStruct((T, AD), _bf16),
            jax.ShapeDtypeStruct((T, AD), _bf16),
            jax.ShapeDtypeStruct((T, AD), _bf16),
            jax.ShapeDtypeStruct((B, E, 128), _f32),
        ],
        compiler_params=pltpu.CompilerParams(
            dimension_semantics=("arbitrary",)),
    )(xf, W_up.astype(_bf16), W_gate.astype(_bf16), W_pre.astype(_bf16),
      W_post.astype(_bf16), W_router,
      A_experts.transpose(2, 0, 1).reshape(AD, E * AD).astype(_bf16),
      g_router.reshape(E, 1), b_router.reshape(E, 1),
      temperature.reshape(1, 1), r2(b_up),
      r2(b_gate), r2(b_pre), r2(b_post), r2(g_adapt), r2(b_adapt),
      g_exp.reshape(1, E * AD), b_exp.reshape(1, E * AD))

    n2 = T // BLK2
    batch_map = lambda i: (i // (S // BLK2), 0)
    y2, rl = pl.pallas_call(
        _stage2_kernel,
        grid=(n2,),
        in_specs=[
            pl.BlockSpec((BLK2, H), row_map),
            pl.BlockSpec((BLK2, D), row_map),
            pl.BlockSpec((S, AD), batch_map),
            pl.BlockSpec((S, AD), batch_map),
            pl.BlockSpec((BLK2, AD), row_map),
            pl.BlockSpec((BLK2, AD), row_map),
            pl.BlockSpec((D, H), c0),
            pl.BlockSpec((D, AD), c0),
            pl.BlockSpec((D, AD), c0),
            pl.BlockSpec((1, D), c0),
            pl.BlockSpec((B, E, 128), c000),
        ],
        out_specs=[
            pl.BlockSpec((BLK2, D), row_map),
            pl.BlockSpec((1, 128), c0),
        ],
        out_shape=[
            jax.ShapeDtypeStruct((T, D), _f32),
            jax.ShapeDtypeStruct((1, 128), _f32),
        ],
        compiler_params=pltpu.CompilerParams(
            dimension_semantics=("arbitrary",)),
    )(hidden, xf, ai, ao, ai, mixed, W_down.astype(_bf16), wda, wcomb,
      r2(b_down), stats)

    return (y2.reshape(B, S, D), rl[0, 0])


# speedup vs baseline: 1.0712x; 1.0712x over previous
import jax
import jax.numpy as jnp
from jax.experimental import pallas as pl
from jax.experimental.pallas import tpu as pltpu

B, S, D = 2, 2048, 1024
E, K = 8, 2
H = 2 * D
AD = H // 16
T = B * S

BLK1 = 1024
BLK2 = 1024

_bf16 = jnp.bfloat16
_f32 = jnp.float32


def _lnk(h, g, b, eps=1e-5):
    m = jnp.mean(h, axis=-1, keepdims=True)
    v = jnp.mean((h - m) * (h - m), axis=-1, keepdims=True)
    return (h - m) * jax.lax.rsqrt(v + eps) * g + b


def _dot_t(a, w):
    return jax.lax.dot_general(a, w, (((1,), (1,)), ((), ())),
                               preferred_element_type=_f32)


def _mm(a, b):
    return jax.lax.dot_general(a, b, (((1,), (0,)), ((), ())),
                               preferred_element_type=_f32)


def _prep_kernel(wup_ref, wgate_ref, wdown_ref, wpre_ref, wpost_ref,
                 wadapt_ref, aexp_ref, wout_ref, wexp_ref,
                 oup_ref, ogate_ref, odown_ref, opre_ref, opost_ref,
                 oacat_ref, owda_ref, owcomb_ref):
    oup_ref[...] = wup_ref[...].astype(_bf16)
    ogate_ref[...] = wgate_ref[...].astype(_bf16)
    wd = wdown_ref[...].astype(_bf16)
    odown_ref[...] = wd
    opre_ref[...] = wpre_ref[...].astype(_bf16)
    opost_ref[...] = wpost_ref[...].astype(_bf16)
    for e in range(E):
        oacat_ref[:, e * AD:(e + 1) * AD] = (
            aexp_ref[e].T.astype(_bf16))
    owda_ref[...] = (0.1 * _mm(wd, wadapt_ref[...].astype(_bf16))
                     ).astype(_bf16)
    owcomb_ref[...] = _mm(wout_ref[...].astype(_bf16),
                          wexp_ref[...].astype(_bf16)).astype(_bf16)


def _stage1_kernel(x_ref, wup_ref, wgate_ref, wpre_ref, wpost_ref, wr_ref,
                   acat_ref, wdown_ref, wcomb_ref, gr_ref, br_ref, temp_ref,
                   bup_ref, bgate_ref, bpre_ref, bpost_ref, ga_ref, ba_ref,
                   gexp_ref, bexp_ref, bdown_ref,
                   y0_ref, ai_ref, ao_ref, stats_ref):
    xb = x_ref[...]
    xbf = xb.astype(_bf16)

    up = _dot_t(xbf, wup_ref[...]) + bup_ref[...]
    gate = _dot_t(xbf, wgate_ref[...]) + bgate_ref[...]
    hidden = jax.nn.silu(gate) * up
    hidden_bf = hidden.astype(_bf16)

    pre = _dot_t(xbf, wpre_ref[...])
    ga = ga_ref[...]
    ba = ba_ref[...]
    adapt_in = _lnk(pre + bpre_ref[...], ga, ba)
    a_ln = _lnk(pre, ga, ba)
    adapt_out = _lnk(_dot_t(hidden_bf, wpost_ref[...]) + bpost_ref[...], ga, ba)
    ai_ref[...] = adapt_in.astype(_bf16)
    ao_ref[...] = adapt_out.astype(_bf16)

    raw_t = jax.lax.dot_general(wr_ref[...], xb, (((1,), (1,)), ((), ())),
                                preferred_element_type=_f32,
                                precision=jax.lax.Precision.HIGHEST)
    m = jnp.mean(raw_t, axis=0, keepdims=True)
    v = jnp.mean((raw_t - m) * (raw_t - m), axis=0, keepdims=True)
    logits_t = (raw_t - m) * jax.lax.rsqrt(v + 1e-5) * gr_ref[...] + br_ref[...]
    z = logits_t / (temp_ref[0, 0] + 1e-6)
    z = z - jnp.max(z, axis=0, keepdims=True)
    ez = jnp.exp(z)
    rw_t = ez / jnp.sum(ez, axis=0, keepdims=True)

    neg = jnp.full((1, BLK1), -jnp.inf, _f32)
    m1 = neg
    m2 = neg
    i1 = jnp.zeros((1, BLK1), jnp.int32)
    i2 = jnp.zeros((1, BLK1), jnp.int32)
    for e in range(E):
        ve = rw_t[e:e + 1, :]
        gt1 = ve > m1
        gt2 = ve > m2
        i2 = jnp.where(gt1, i1, jnp.where(gt2, e, i2))
        m2 = jnp.where(gt1, m1, jnp.where(gt2, ve, m2))
        i1 = jnp.where(gt1, e, i1)
        m1 = jnp.where(gt1, ve, m1)
    eids = jax.lax.broadcasted_iota(jnp.int32, (E, BLK1), 0)
    dispatch_t = (jnp.where(eids == i1, m1, 0.0)
                  + jnp.where(eids == i2, m2, 0.0))

    r8 = (jax.lax.broadcasted_iota(jnp.int32, (E, E * AD), 1) // AD
          == jax.lax.broadcasted_iota(jnp.int32, (E, E * AD), 0))
    r8 = r8.astype(_bf16)
    mavg = (jax.lax.broadcasted_iota(jnp.int32, (E * AD, E), 0) // AD
            == jax.lax.broadcasted_iota(jnp.int32, (E * AD, E), 1))
    mavg = mavg.astype(_bf16) * (1.0 / AD)
    rsum = (jax.lax.broadcasted_iota(jnp.int32, (E * AD, AD), 0) % AD
            == jax.lax.broadcasted_iota(jnp.int32, (E * AD, AD), 1))
    rsum = rsum.astype(_bf16)

    abf = a_ln.astype(_bf16)
    h_all = _mm(abf, acat_ref[...])
    mean8 = _mm(h_all.astype(_bf16), mavg)
    ex28 = _mm((h_all * h_all).astype(_bf16), mavg)
    mean_full = _mm(mean8.astype(_bf16), r8)
    ex2_full = _mm(ex28.astype(_bf16), r8)
    rinv = jax.lax.rsqrt(jnp.maximum(ex2_full - mean_full * mean_full, 0.0)
                         + 1e-5)
    hl_all = (h_all - mean_full) * rinv * gexp_ref[...] + bexp_ref[...]
    dfull = jax.lax.dot_general(dispatch_t.astype(_bf16), r8,
                                (((0,), (0,)), ((), ())),
                                preferred_element_type=_f32)
    mixed = _mm((hl_all * dfull).astype(_bf16), rsum)

    y0_ref[...] = (xb + bdown_ref[...]
                   + _dot_t(hidden_bf, wdown_ref[...])
                   + _dot_t(mixed.astype(_bf16), wcomb_ref[...]))

    col8 = jnp.sum(dispatch_t, axis=1, keepdims=True)
    zp = jnp.sum(logits_t * logits_t)
    li = jax.lax.broadcasted_iota(jnp.int32, (E, 128), 1)
    si = jax.lax.broadcasted_iota(jnp.int32, (E, 128), 0)
    srow = (jnp.where(li == 0, col8, 0.0)
            + jnp.where((li == 1) & (si == 0), zp, 0.0)).reshape(1, E, 128)
    i = pl.program_id(0)

    @pl.when(i % (S // BLK1) == 0)
    def _init():
        stats_ref[...] = srow

    @pl.when(i % (S // BLK1) != 0)
    def _acc():
        stats_ref[...] += srow


def _stage2_kernel(y0_ref, aiall_ref, aoall_ref, aib_ref, wda_ref, stats_ref,
                   y_ref, rloss_ref):
    aw = _dot_t(aib_ref[...], aoall_ref[...]).astype(_bf16)
    aw = jax.nn.silu(jnp.clip(aw, _bf16(-5.0), _bf16(5.0)))
    adapt = jax.lax.dot_general(aw, aiall_ref[...],
                                (((1,), (0,)), ((), ())),
                                preferred_element_type=_f32)
    y_ref[...] = y0_ref[...] + _dot_t(adapt.astype(_bf16), wda_ref[...])

    @pl.when(pl.program_id(0) == 0)
    def _loss():
        st = stats_ref[...]
        loads = st[:, :, 0] * (1.0 / S)
        zsum = jnp.sum(st[:, 0:1, 1:2])
        mean_l = jnp.mean(loads)
        var = jnp.sum((loads - mean_l) * (loads - mean_l)) / (B * E - 1)
        lb = jnp.sqrt(var) / mean_l * 10.0
        zl = zsum * (1.0 / (T * E))
        val = 0.001 * zl + 0.1 * lb
        rloss_ref[...] = jnp.full((1, 128), val, _f32)


def kernel(x, W_router, g_router, b_router, temperature, W_up, b_up, W_gate,
           b_gate, W_down, b_down, W_pre, b_pre, W_post, b_post, g_adapt,
           b_adapt, W_adapt_proj, A_experts, g_exp, b_exp, W_expert_proj,
           W_output_proj):
    xf = x.reshape(T, D)
    r2 = lambda v: v.reshape(1, -1)

    wup, wgate, wdown, wpre, wpost, acat, wda, wcomb = pl.pallas_call(
        _prep_kernel,
        out_shape=[
            jax.ShapeDtypeStruct((H, D), _bf16),
            jax.ShapeDtypeStruct((H, D), _bf16),
            jax.ShapeDtypeStruct((D, H), _bf16),
            jax.ShapeDtypeStruct((AD, D), _bf16),
            jax.ShapeDtypeStruct((AD, H), _bf16),
            jax.ShapeDtypeStruct((AD, E * AD), _bf16),
            jax.ShapeDtypeStruct((D, AD), _bf16),
            jax.ShapeDtypeStruct((D, AD), _bf16),
        ],
    )(W_up, W_gate, W_down, W_pre, W_post, W_adapt_proj, A_experts,
      W_output_proj, W_expert_proj)

    n1 = T // BLK1
    c0 = lambda i: (0, 0)
    c000 = lambda i: (0, 0, 0)
    row_map = lambda i: (i, 0)
    y0, ai, ao, stats = pl.pallas_call(
        _stage1_kernel,
        grid=(n1,),
        in_specs=[
            pl.BlockSpec((BLK1, D), row_map),
            pl.BlockSpec((H, D), c0),
            pl.BlockSpec((H, D), c0),
            pl.BlockSpec((AD, D), c0),
            pl.BlockSpec((AD, H), c0),
            pl.BlockSpec((E, D), c0),
            pl.BlockSpec((AD, E * AD), c0),
            pl.BlockSpec((D, H), c0),
            pl.BlockSpec((D, AD), c0),
            pl.BlockSpec((E, 1), c0),
            pl.BlockSpec((E, 1), c0),
            pl.BlockSpec((1, 1), c0),
            pl.BlockSpec((1, H), c0),
            pl.BlockSpec((1, H), c0),
            pl.BlockSpec((1, AD), c0),
            pl.BlockSpec((1, AD), c0),
            pl.BlockSpec((1, AD), c0),
            pl.BlockSpec((1, AD), c0),
            pl.BlockSpec((1, E * AD), c0),
            pl.BlockSpec((1, E * AD), c0),
            pl.BlockSpec((1, D), c0),
        ],
        out_specs=[
            pl.BlockSpec((BLK1, D), row_map),
            pl.BlockSpec((BLK1, AD), row_map),
            pl.BlockSpec((BLK1, AD), row_map),
            pl.BlockSpec((1, E, 128), lambda i: (i // (S // BLK1), 0, 0)),
        ],
        out_shape=[
            jax.ShapeDtypeStruct((T, D), _f32),
            jax.ShapeDtypeStruct((T, AD), _bf16),
            jax.ShapeDtypeStruct((T, AD), _bf16),
            jax.ShapeDtypeStruct((B, E, 128), _f32),
        ],
        compiler_params=pltpu.CompilerParams(
            dimension_semantics=("arbitrary",)),
    )(xf, wup, wgate, wpre, wpost, W_router, acat, wdown, wcomb,
      g_router.reshape(E, 1), b_router.reshape(E, 1),
      temperature.reshape(1, 1), r2(b_up), r2(b_gate), r2(b_pre), r2(b_post),
      r2(g_adapt), r2(b_adapt), g_exp.reshape(1, E * AD),
      b_exp.reshape(1, E * AD), r2(b_down))

    n2 = T // BLK2
    batch_map = lambda i: (i // (S // BLK2), 0)
    y2, rl = pl.pallas_call(
        _stage2_kernel,
        grid=(n2,),
        in_specs=[
            pl.BlockSpec((BLK2, D), row_map),
            pl.BlockSpec((S, AD), batch_map),
            pl.BlockSpec((S, AD), batch_map),
            pl.BlockSpec((BLK2, AD), row_map),
            pl.BlockSpec((D, AD), c0),
            pl.BlockSpec((B, E, 128), c000),
        ],
        out_specs=[
            pl.BlockSpec((BLK2, D), row_map),
            pl.BlockSpec((1, 128), c0),
        ],
        out_shape=[
            jax.ShapeDtypeStruct((T, D), _f32),
            jax.ShapeDtypeStruct((1, 128), _f32),
        ],
        compiler_params=pltpu.CompilerParams(
            dimension_semantics=("arbitrary",)),
    )(y0, ai, ao, ai, wda, stats)

    return (y2.reshape(B, S, D), rl[0, 0])
